# Initial kernel scaffold; baseline (speedup 1.0000x reference)
import jax
import jax.numpy as jnp
from jax.experimental import pallas as pl


def kernel(x, target):
    raise NotImplementedError("write your pallas kernel here")


# speedup vs baseline: 6.5563x; 6.5563x over previous
import math

import jax
import jax.numpy as jnp
from jax import lax
from jax.experimental import pallas as pl
from jax.experimental.pallas import tpu as pltpu

_N = 2048
_SIZE = 32000
_CONF = 0.9
_FILL = 0.1 / (_SIZE - 2)
_C = _CONF * math.log(_CONF) + 0.1 * math.log(_FILL)

_BR = 256
_BC = 6400
_GR = _N // _BR
_GC = _SIZE // _BC


def _body(tgt_ref, x_ref, out_ref):
    i = pl.program_id(0)
    j = pl.program_id(1)

    x = x_ref[...]
    tgt = tgt_ref[0]

    col = lax.broadcasted_iota(jnp.int32, (_BR, _BC), 1) + j * _BC
    coef = jnp.where(
        tgt == 0,
        0.0,
        jnp.where(col == 0, 0.0, jnp.where(col == tgt, -_CONF, -_FILL)),
    ).astype(jnp.float32)
    partial = jnp.sum(x * coef)

    @pl.when((i == 0) & (j == 0))
    def _init():
        out_ref[0, 0] = 0.0

    @pl.when(j == 0)
    def _row_const():
        cnt = jnp.sum((tgt != 0).astype(jnp.float32))
        out_ref[0, 0] += _C * cnt

    out_ref[0, 0] += partial


def kernel(x, target):
    tgt3 = target.reshape(_GR, _BR, 1)
    out = pl.pallas_call(
        _body,
        grid=(_GR, _GC),
        in_specs=[
            pl.BlockSpec((1, _BR, 1), lambda i, j: (i, 0, 0)),
            pl.BlockSpec((_BR, _BC), lambda i, j: (i, j)),
        ],
        out_specs=pl.BlockSpec(
            (1, 1), lambda i, j: (0, 0), memory_space=pltpu.SMEM
        ),
        out_shape=jax.ShapeDtypeStruct((1, 1), jnp.float32),
    )(tgt3, x)
    return out[0, 0]
